# Initial kernel scaffold; baseline (speedup 1.0000x reference)
import jax
import jax.numpy as jnp
from jax.experimental import pallas as pl


def kernel(x, var_emb):
    raise NotImplementedError("write your pallas kernel here")


# speedup vs baseline: 4.0937x; 4.0937x over previous
import functools

import jax
import jax.numpy as jnp
from jax import lax
from jax.experimental import pallas as pl
from jax.experimental.pallas import tpu as pltpu
from jax.experimental.pallas import tpu_sc as plsc

_NC = 2
_NS = 16
_NW = _NC * _NS
_REP = 8


def kernel(x, var_emb):
    B, L, D = x.shape
    V, M = var_emb.shape
    N = B * L
    NG = N // _REP
    per_w = NG // _NW

    mesh = plsc.VectorSubcoreMesh(core_axis_name="c", subcore_axis_name="s")

    @functools.partial(
        pl.kernel,
        mesh=mesh,
        out_type=jax.ShapeDtypeStruct((NG, _REP, D, M), jnp.float32),
        scratch_types=[
            pltpu.VMEM((_REP, D, M), jnp.float32),
            pltpu.SemaphoreType.DMA,
        ],
    )
    def emb_broadcast(emb_hbm, out_hbm, stage_v, sem):
        wid = lax.axis_index("s") * _NC + lax.axis_index("c")
        for r in range(_REP):
            pltpu.sync_copy(emb_hbm.at[pl.ds(0, D)], stage_v.at[r])
        base = wid * per_w
        copies = [
            pltpu.async_copy(stage_v, out_hbm.at[base + i], sem)
            for i in range(per_w)
        ]
        for cp in copies:
            cp.wait()

    out = emb_broadcast(var_emb)
    return out.reshape(B, L, D, M)


# speedup vs baseline: 4.1652x; 1.0175x over previous
import functools

import jax
import jax.numpy as jnp
from jax import lax
from jax.experimental import pallas as pl
from jax.experimental.pallas import tpu as pltpu
from jax.experimental.pallas import tpu_sc as plsc

_NC = 2
_NS = 16
_NW = _NC * _NS
_REP = 8


def kernel(x, var_emb):
    B, L, D = x.shape
    V, M = var_emb.shape
    N = B * L
    NG = N // _REP
    per_w = NG // _NW

    mesh = plsc.VectorSubcoreMesh(core_axis_name="c", subcore_axis_name="s")

    @functools.partial(
        pl.kernel,
        mesh=mesh,
        out_type=jax.ShapeDtypeStruct((NG, _REP, D, M), jnp.float32),
        scratch_types=[
            pltpu.VMEM((_REP, D, M), jnp.float32),
            pltpu.SemaphoreType.DMA,
            pltpu.SemaphoreType.DMA,
        ],
    )
    def emb_broadcast(emb_hbm, out_hbm, stage_v, sem_in, sem_out):
        wid = lax.axis_index("s") * _NC + lax.axis_index("c")
        loads = [
            pltpu.async_copy(emb_hbm.at[pl.ds(0, D)], stage_v.at[r], sem_in)
            for r in range(_REP)
        ]
        for cp in loads:
            cp.wait()
        base = wid * per_w
        copies = [
            pltpu.async_copy(stage_v, out_hbm.at[base + i], sem_out)
            for i in range(per_w)
        ]
        for cp in copies:
            cp.wait()

    out = emb_broadcast(var_emb)
    return out.reshape(B, L, D, M)
